# Initial kernel scaffold; baseline (speedup 1.0000x reference)
import jax
import jax.numpy as jnp
from jax.experimental import pallas as pl


def kernel(x, batch, edge_index, W_fe, b_fe, W0, b0, W1a, b1a, W1b, b1b, W2a, b2a, W2b, b2b, Wf1, bf1, Wf2, bf2):
    raise NotImplementedError("write your pallas kernel here")


# speedup vs baseline: 1.8795x; 1.8795x over previous
import functools

import jax
import jax.numpy as jnp
from jax.experimental import pallas as pl
from jax.experimental.pallas import tpu as pltpu


def _leaky(v):
    return jnp.where(v > 0, v, 0.1 * v)


def _node_stage_kernel(feat_ref, kf_ref, w0f_ref, w0k_ref, b0c_ref,
                       a_ref, b_ref, r_ref, ba_ref,
                       h_ref, p_ref, q_ref):
    feat = feat_ref[...]
    kf = kf_ref[...]
    h = feat @ w0f_ref[...] + kf * w0k_ref[...] + b0c_ref[...]
    h = jnp.maximum(h, 0.0)
    kr = kf * r_ref[...]
    h_ref[...] = h
    p_ref[...] = h @ a_ref[...] + kr
    q_ref[...] = h @ b_ref[...] - kr + ba_ref[...]


def _node_stage(feat, kf, w0f, w0k, b0c, A, B, r, ba, R=2048):
    n = feat.shape[0]
    grid = (n // R,)
    full = lambda s: pl.BlockSpec(s, lambda i: (0, 0))
    row = lambda w: pl.BlockSpec((R, w), lambda i: (i, 0))
    return pl.pallas_call(
        _node_stage_kernel,
        grid=grid,
        in_specs=[row(feat.shape[1]), row(1), full(w0f.shape), full(w0k.shape),
                  full(b0c.shape), full(A.shape), full(B.shape), full(r.shape),
                  full(ba.shape)],
        out_specs=[row(64), row(64), row(64)],
        out_shape=[jax.ShapeDtypeStruct((n, 64), jnp.float32)] * 3,
    )(feat, kf, w0f, w0k, b0c, A, B, r, ba)


def _update_stage_kernel(agg_ref, res_ref, kf_ref, a_ref, b_ref, r_ref, ba_ref,
                         x_ref, p_ref, q_ref):
    agg = agg_ref[...]
    agg = jnp.where(jnp.isfinite(agg), agg, 0.0)
    x = jnp.maximum(agg + res_ref[...], 0.0)
    kr = kf_ref[...] * r_ref[...]
    x_ref[...] = x
    p_ref[...] = x @ a_ref[...] + kr
    q_ref[...] = x @ b_ref[...] - kr + ba_ref[...]


def _update_stage(agg, res, kf, A, B, r, ba, R=2048):
    n = agg.shape[0]
    grid = (n // R,)
    full = lambda s: pl.BlockSpec(s, lambda i: (0, 0))
    row = lambda w: pl.BlockSpec((R, w), lambda i: (i, 0))
    return pl.pallas_call(
        _update_stage_kernel,
        grid=grid,
        in_specs=[row(64), row(64), row(1), full(A.shape), full(B.shape),
                  full(r.shape), full(ba.shape)],
        out_specs=[row(64), row(64), row(64)],
        out_shape=[jax.ShapeDtypeStruct((n, 64), jnp.float32)] * 3,
    )(agg, res, kf, A, B, r, ba)


def _edge_mm_kernel(g_ref, wb_ref, bb_ref, m_ref):
    m_ref[...] = _leaky(g_ref[...]) @ wb_ref[...] + bb_ref[...]


def _edge_mm(G, Wb, bb, R=4096):
    n = G.shape[0]
    return pl.pallas_call(
        _edge_mm_kernel,
        grid=(n // R,),
        in_specs=[pl.BlockSpec((R, 64), lambda i: (i, 0)),
                  pl.BlockSpec(Wb.shape, lambda i: (0, 0)),
                  pl.BlockSpec(bb.shape, lambda i: (0, 0))],
        out_specs=pl.BlockSpec((R, 64), lambda i: (i, 0)),
        out_shape=jax.ShapeDtypeStruct((n, 64), jnp.float32),
    )(G, Wb, bb)


def _pool_head_kernel(agg2_ref, x1_ref, batch_ref, wf1_ref, bf1_ref,
                      wf2_ref, bf2_ref, out_ref,
                      maxacc, sumacc, cntacc, *, n_valid, R, num_graphs):
    pid = pl.program_id(0)
    nsteps = pl.num_programs(0)

    @pl.when(pid == 0)
    def _init():
        maxacc[...] = jnp.full_like(maxacc, -jnp.inf)
        sumacc[...] = jnp.zeros_like(sumacc)
        cntacc[...] = jnp.zeros_like(cntacc)

    agg2 = agg2_ref[...]
    agg2 = jnp.where(jnp.isfinite(agg2), agg2, 0.0)
    x1 = x1_ref[...]
    x2 = jnp.maximum(agg2 + x1, 0.0)
    cat = jnp.concatenate([x1, x2], axis=1)

    batch = batch_ref[...]
    rowid = jax.lax.broadcasted_iota(jnp.int32, (R, 1), 0)
    valid = (pid * R + rowid) < n_valid

    gid = jax.lax.broadcasted_iota(jnp.int32, (R, num_graphs), 1)
    onehot = jnp.where((batch == gid) & valid, 1.0, 0.0)
    sumacc[...] += jax.lax.dot_general(onehot, cat, (((0,), (0,)), ((), ())))
    cntacc[...] += jnp.sum(onehot, axis=0, keepdims=True)

    g0 = batch[0, 0]
    g1 = batch[R - 1, 0]

    def body(g, _):
        sel = (batch == g) & valid
        vals = jnp.where(sel, cat, -jnp.inf)
        m = jnp.max(vals, axis=0, keepdims=True)
        cur = maxacc[pl.ds(g, 1), :]
        maxacc[pl.ds(g, 1), :] = jnp.maximum(cur, m)
        return 0

    jax.lax.fori_loop(g0, g1 + 1, body, 0)

    @pl.when(pid == nsteps - 1)
    def _final():
        mp = maxacc[...]
        mp = jnp.where(jnp.isfinite(mp), mp, 0.0)
        cnt = jnp.maximum(cntacc[...], 1.0)
        mean = sumacc[...] / cnt.reshape(num_graphs, 1)
        feat = jnp.concatenate([mp, mean], axis=1)
        o = jnp.maximum(feat @ wf1_ref[...] + bf1_ref[...], 0.0)
        o = o @ wf2_ref[...] + bf2_ref[...]
        lse = jnp.log(jnp.sum(jnp.exp(o - jnp.max(o, axis=1, keepdims=True)),
                              axis=1, keepdims=True)) + jnp.max(o, axis=1, keepdims=True)
        out_ref[...] = o - lse


def _pool_head(agg2, x1, batch2d, Wf1, bf1, Wf2, bf2, n_valid, num_graphs, R=2048):
    n = agg2.shape[0]
    full = lambda s: pl.BlockSpec(s, lambda i: (0, 0))
    row = lambda w: pl.BlockSpec((R, w), lambda i: (i, 0))
    kern = functools.partial(_pool_head_kernel, n_valid=n_valid, R=R,
                             num_graphs=num_graphs)
    return pl.pallas_call(
        kern,
        grid=(n // R,),
        in_specs=[row(64), row(64), row(1), full(Wf1.shape), full(bf1.shape),
                  full(Wf2.shape), full(bf2.shape)],
        out_specs=pl.BlockSpec((num_graphs, 2), lambda i: (0, 0)),
        out_shape=jax.ShapeDtypeStruct((num_graphs, 2), jnp.float32),
        scratch_shapes=[pltpu.VMEM((num_graphs, 128), jnp.float32),
                        pltpu.VMEM((num_graphs, 128), jnp.float32),
                        pltpu.VMEM((1, num_graphs), jnp.float32)],
    )(agg2, x1, batch2d, Wf1, bf1, Wf2, bf2)


def kernel(x, batch, edge_index, W_fe, b_fe, W0, b0, W1a, b1a, W1b, b1b,
           W2a, b2a, W2b, b2b, Wf1, bf1, Wf2, bf2):
    n = x.shape[0]
    H = W0.shape[1]
    feat_dim = x.shape[1] - 1

    R = 2048
    n_pad = ((n + R - 1) // R) * R

    kf = x[:, 0:1]
    feat = x[:, 1:]
    kf_p = jnp.pad(kf, ((0, n_pad - n), (0, 0)))
    feat_p = jnp.pad(feat, ((0, n_pad - n), (0, 0)))
    batch_p = jnp.pad(batch.reshape(n, 1), ((0, n_pad - n), (0, 0)), mode='edge')

    w0f = W0[:feat_dim]
    w0k = W_fe @ W0[feat_dim:]
    b0c = (b_fe @ W0[feat_dim:] + b0).reshape(1, H)

    A1, B1, r1 = W1a[:H], W1a[H:2 * H] - W1a[:H], W1a[2 * H:2 * H + 1]
    A2, B2, r2 = W2a[:H], W2a[H:2 * H] - W2a[:H], W2a[2 * H:2 * H + 1]

    h, P1, Q1 = _node_stage(feat_p, kf_p, w0f, w0k, b0c, A1, B1, r1,
                            b1a.reshape(1, H))

    src = edge_index[0]
    dst = edge_index[1]

    def conv(P, Q, Wb, bb):
        G = jnp.take(P, src, axis=0) + jnp.take(Q, dst, axis=0)
        e = G.shape[0]
        e_pad = ((e + 4095) // 4096) * 4096
        G = jnp.pad(G, ((0, e_pad - e), (0, 0)))
        M = _edge_mm(G, Wb, bb.reshape(1, H))[:e]
        return jax.ops.segment_max(M, dst, num_segments=n_pad)

    agg1 = conv(P1, Q1, W1b, b1b)
    x1, P2, Q2 = _update_stage(agg1, h, kf_p, A2, B2, r2, b2a.reshape(1, H))
    agg2 = conv(P2, Q2, W2b, b2b)

    return _pool_head(agg2, x1, batch_p, Wf1, bf1.reshape(1, H),
                      Wf2, bf2.reshape(1, 2), n, 64)
